# Initial kernel scaffold; baseline (speedup 1.0000x reference)
import functools

import jax
import jax.numpy as jnp
from jax import lax
from jax.experimental import pallas as pl
from jax.experimental.pallas import tpu as pltpu
from jax.experimental.pallas import tpu_sc as plsc

N = 10000
E = 320000
IN_DIM = 128
H = 8
D = 16
NEG_SLOPE = 0.2

NC = 2
NS = 16
NW = NC * NS
EW = E // NW
K = 200
NCHUNK = EW // K
RPT = N // NS
RZB = 125
TW = IN_DIM + 16

TCB = 1000


def _tc_tables_body(h_ref, wsl_ref, wr_ref, tbl_ref, er_ref):
    x = h_ref[...]
    tbl_ref[...] = jnp.dot(x, wsl_ref[...], preferred_element_type=jnp.float32)
    er_ref[...] = jnp.dot(x, wr_ref[...], preferred_element_type=jnp.float32)


def _sc_edge_body(tbl_hbm, er_hbm, src_hbm, dst_hbm, part_hbm,
                  src_v, dst_v, tbl_buf, er_buf, zbuf, acc, gsem, ssem):
    c = lax.axis_index("c")
    s = lax.axis_index("s")
    wid = s * NC + c

    pltpu.sync_copy(src_hbm.at[wid], src_v)
    pltpu.sync_copy(dst_hbm.at[wid], dst_v)

    zeros16 = jnp.zeros((16,), jnp.float32)

    def zrow(i, _):
        for j in range(TW // 16):
            zbuf[i, pl.ds(j * 16, 16)] = zeros16
        return 0

    lax.fori_loop(0, RZB, zrow, 0)
    for q in range(RPT // RZB):
        pltpu.sync_copy(zbuf, acc.at[pl.ds(s * RPT + q * RZB, RZB)])
    plsc.subcore_barrier()

    def chunk(cidx, _):
        cp1 = pltpu.async_copy(tbl_hbm.at[src_v.at[cidx]], tbl_buf, gsem)
        cp2 = pltpu.async_copy(er_hbm.at[dst_v.at[cidx]], er_buf, ssem)
        cp1.wait()
        cp2.wait()

        def edge(e, _):
            el = tbl_buf[e, pl.ds(IN_DIM, 16)]
            er = er_buf[e, :]
            x = el + er
            w = jnp.exp(jnp.maximum(x, NEG_SLOPE * x))
            for hh in range(H):
                wh = jnp.take(w, jnp.full((16,), hh, jnp.int32),
                              mode=lax.GatherScatterMode.PROMISE_IN_BOUNDS)
                f = tbl_buf[e, pl.ds(hh * 16, 16)]
                tbl_buf[e, pl.ds(hh * 16, 16)] = f * wh
            tbl_buf[e, pl.ds(IN_DIM, 16)] = w
            return 0

        lax.fori_loop(0, K, edge, 0)
        pltpu.sync_copy(tbl_buf, acc.at[dst_v.at[cidx]], add=True)
        return 0

    lax.fori_loop(0, NCHUNK, chunk, 0)

    plsc.subcore_barrier()
    for q in range(RPT // RZB):
        r0 = s * RPT + q * RZB
        pltpu.sync_copy(acc.at[pl.ds(r0, RZB)], part_hbm.at[c, pl.ds(r0, RZB)])


def _tc_final_body(h_ref, p0_ref, p1_ref, t_ref, o_ref):
    p = p0_ref[0] + p1_ref[0]
    num = p[:, :IN_DIM]
    den = p[:, IN_DIM:]
    den_exp = jnp.dot(den, t_ref[...], preferred_element_type=jnp.float32)
    r = num / (den_exp + 1e-9)
    o_ref[...] = h_ref[...] + jnp.where(r > 0, r, jnp.expm1(r))


@jax.jit
def kernel(h, edge_index, W, attn_l, attn_r):
    f32 = jnp.float32
    W3 = W.reshape(IN_DIM, H, D)
    Wl = (W3 * attn_l[None]).sum(-1)
    Wr = (W3 * attn_r[None]).sum(-1)
    pad = jnp.zeros((IN_DIM, 16 - H), f32)
    Wsl = jnp.concatenate([W, Wl, pad], axis=1)
    Wr16 = jnp.concatenate([Wr, pad], axis=1)

    src = edge_index[0].astype(jnp.int32).reshape(NW, NCHUNK, K)
    dst = edge_index[1].astype(jnp.int32).reshape(NW, NCHUNK, K)

    tbl, er16 = pl.pallas_call(
        _tc_tables_body,
        grid=(N // TCB,),
        in_specs=[
            pl.BlockSpec((TCB, IN_DIM), lambda i: (i, 0)),
            pl.BlockSpec((IN_DIM, TW), lambda i: (0, 0)),
            pl.BlockSpec((IN_DIM, 16), lambda i: (0, 0)),
        ],
        out_specs=[
            pl.BlockSpec((TCB, TW), lambda i: (i, 0)),
            pl.BlockSpec((TCB, 16), lambda i: (i, 0)),
        ],
        out_shape=[
            jax.ShapeDtypeStruct((N, TW), f32),
            jax.ShapeDtypeStruct((N, 16), f32),
        ],
    )(h, Wsl, Wr16)

    mesh = plsc.VectorSubcoreMesh(core_axis_name="c", subcore_axis_name="s")
    part = pl.kernel(
        _sc_edge_body,
        out_type=jax.ShapeDtypeStruct((NC, N, TW), f32),
        mesh=mesh,
        scratch_types=[
            pltpu.VMEM((NCHUNK, K), jnp.int32),
            pltpu.VMEM((NCHUNK, K), jnp.int32),
            pltpu.VMEM((K, TW), f32),
            pltpu.VMEM((K, 16), f32),
            pltpu.VMEM((RZB, TW), f32),
            pltpu.VMEM_SHARED((N, TW), f32),
            pltpu.SemaphoreType.DMA,
            pltpu.SemaphoreType.DMA,
        ],
    )(tbl, er16, src, dst)

    T = (jnp.arange(128)[None, :] // D == jnp.arange(16)[:, None]).astype(f32)
    out = pl.pallas_call(
        _tc_final_body,
        grid=(N // TCB,),
        in_specs=[
            pl.BlockSpec((TCB, IN_DIM), lambda i: (i, 0)),
            pl.BlockSpec((1, TCB, TW), lambda i: (0, i, 0)),
            pl.BlockSpec((1, TCB, TW), lambda i: (1, i, 0)),
            pl.BlockSpec((16, IN_DIM), lambda i: (0, 0)),
        ],
        out_specs=pl.BlockSpec((TCB, IN_DIM), lambda i: (i, 0)),
        out_shape=jax.ShapeDtypeStruct((N, IN_DIM), f32),
    )(h, part, part, T)
    return out


# speedup vs baseline: 82.1667x; 82.1667x over previous
import functools

import jax
import jax.numpy as jnp
from jax import lax
from jax.experimental import pallas as pl
from jax.experimental.pallas import tpu as pltpu
from jax.experimental.pallas import tpu_sc as plsc

N = 10000
E = 320000
IN_DIM = 128
H = 8
D = 16
NEG_SLOPE = 0.2

NC = 2
NS = 16
NW = NC * NS
EW = E // NW
K = 200
NCHUNK = EW // K
RPT = N // NS
RZB = 125
TW = IN_DIM + 16

TCB = 1000


def _tc_tables_body(h_ref, wsl_ref, wr_ref, tbl_ref, er_ref):
    x = h_ref[...]
    tbl_ref[...] = jnp.dot(x, wsl_ref[...], preferred_element_type=jnp.float32)
    er_ref[...] = jnp.dot(x, wr_ref[...], preferred_element_type=jnp.float32)


def _sc_edge_body(tbl_hbm, er_hbm, src_hbm, dst_hbm, part_hbm,
                  src_c, dst_c, tbl_buf, er_buf, acc, gsem, ssem):
    c = lax.axis_index("c")
    s = lax.axis_index("s")
    wid = s * NC + c

    zeros16 = jnp.zeros((16,), jnp.float32)

    def zrow(i, _):
        for j in range(TW // 16):
            tbl_buf[i, pl.ds(j * 16, 16)] = zeros16
        return 0

    lax.fori_loop(0, RZB, zrow, 0)
    for q in range(RPT // RZB):
        pltpu.sync_copy(tbl_buf.at[pl.ds(0, RZB)],
                        acc.at[pl.ds(s * RPT + q * RZB, RZB)])
    plsc.subcore_barrier()

    def chunk(cidx, _):
        pltpu.sync_copy(src_hbm.at[wid, cidx], src_c)
        pltpu.sync_copy(dst_hbm.at[wid, cidx], dst_c)
        cp1 = pltpu.async_copy(tbl_hbm.at[src_c], tbl_buf, gsem)
        cp2 = pltpu.async_copy(er_hbm.at[dst_c], er_buf, ssem)
        cp1.wait()
        cp2.wait()

        def edge(e, _):
            el = tbl_buf[e, pl.ds(IN_DIM, 16)]
            er = er_buf[e, :]
            x = el + er
            w = jnp.exp(jnp.maximum(x, NEG_SLOPE * x))
            dnums = lax.GatherDimensionNumbers(
                offset_dims=(), collapsed_slice_dims=(0,), start_index_map=(0,))
            for hh in range(H):
                wh = lax.gather(w, jnp.full((16, 1), hh, jnp.int32), dnums,
                                slice_sizes=(1,),
                                mode=lax.GatherScatterMode.PROMISE_IN_BOUNDS)
                f = tbl_buf[e, pl.ds(hh * 16, 16)]
                tbl_buf[e, pl.ds(hh * 16, 16)] = f * wh
            tbl_buf[e, pl.ds(IN_DIM, 16)] = w
            return 0

        lax.fori_loop(0, K, edge, 0)
        pltpu.sync_copy(tbl_buf, acc.at[dst_c], add=True)
        return 0

    lax.fori_loop(0, NCHUNK, chunk, 0)

    plsc.subcore_barrier()
    for q in range(RPT // RZB):
        r0 = s * RPT + q * RZB
        pltpu.sync_copy(acc.at[pl.ds(r0, RZB)], part_hbm.at[c, pl.ds(r0, RZB)])


def _tc_final_body(h_ref, p0_ref, p1_ref, t_ref, o_ref):
    p = p0_ref[0] + p1_ref[0]
    num = p[:, :IN_DIM]
    den = p[:, IN_DIM:]
    den_exp = jnp.dot(den, t_ref[...], preferred_element_type=jnp.float32)
    r = num / (den_exp + 1e-9)
    o_ref[...] = h_ref[...] + jnp.where(r > 0, r, jnp.exp(r) - 1.0)


@jax.jit
def kernel(h, edge_index, W, attn_l, attn_r):
    f32 = jnp.float32
    W3 = W.reshape(IN_DIM, H, D)
    Wl = (W3 * attn_l[None]).sum(-1)
    Wr = (W3 * attn_r[None]).sum(-1)
    pad = jnp.zeros((IN_DIM, 16 - H), f32)
    Wsl = jnp.concatenate([W, Wl, pad], axis=1)
    Wr16 = jnp.concatenate([Wr, pad], axis=1)

    src = edge_index[0].astype(jnp.int32).reshape(NW, NCHUNK, K)
    dst = edge_index[1].astype(jnp.int32).reshape(NW, NCHUNK, K)

    tbl, er16 = pl.pallas_call(
        _tc_tables_body,
        grid=(N // TCB,),
        in_specs=[
            pl.BlockSpec((TCB, IN_DIM), lambda i: (i, 0)),
            pl.BlockSpec((IN_DIM, TW), lambda i: (0, 0)),
            pl.BlockSpec((IN_DIM, 16), lambda i: (0, 0)),
        ],
        out_specs=[
            pl.BlockSpec((TCB, TW), lambda i: (i, 0)),
            pl.BlockSpec((TCB, 16), lambda i: (i, 0)),
        ],
        out_shape=[
            jax.ShapeDtypeStruct((N, TW), f32),
            jax.ShapeDtypeStruct((N, 16), f32),
        ],
    )(h, Wsl, Wr16)

    mesh = plsc.VectorSubcoreMesh(core_axis_name="c", subcore_axis_name="s")
    part = pl.kernel(
        _sc_edge_body,
        out_type=jax.ShapeDtypeStruct((NC, N, TW), f32),
        mesh=mesh,
        scratch_types=[
            pltpu.VMEM((K,), jnp.int32),
            pltpu.VMEM((K,), jnp.int32),
            pltpu.VMEM((K, TW), f32),
            pltpu.VMEM((K, 16), f32),
            pltpu.VMEM_SHARED((N, TW), f32),
            pltpu.SemaphoreType.DMA,
            pltpu.SemaphoreType.DMA,
        ],
        compiler_params=pltpu.CompilerParams(use_tc_tiling_on_sc=False),
    )(tbl, er16, src, dst)

    T = (jnp.arange(128)[None, :] // D == jnp.arange(16)[:, None]).astype(f32)
    out = pl.pallas_call(
        _tc_final_body,
        grid=(N // TCB,),
        in_specs=[
            pl.BlockSpec((TCB, IN_DIM), lambda i: (i, 0)),
            pl.BlockSpec((1, TCB, TW), lambda i: (0, i, 0)),
            pl.BlockSpec((1, TCB, TW), lambda i: (1, i, 0)),
            pl.BlockSpec((16, IN_DIM), lambda i: (0, 0)),
        ],
        out_specs=pl.BlockSpec((TCB, IN_DIM), lambda i: (i, 0)),
        out_shape=jax.ShapeDtypeStruct((N, IN_DIM), f32),
    )(h, part, part, T)
    return out
